# baseline (speedup 1.0000x reference)
import functools

import jax
import jax.numpy as jnp
from jax import lax
from jax.experimental import pallas as pl
from jax.experimental.pallas import tpu as pltpu
from jax.experimental.pallas import tpu_sc as plsc

_D = 110
_DP = 112
_CHUNK = 128


@functools.lru_cache(maxsize=None)
def _build_gather(B: int, V: int):
    info = plsc.get_sparse_core_info()
    NC, NS = info.num_cores, info.num_subcores
    NW = NC * NS
    assert B % (NW * _CHUNK) == 0
    b_per_w = B // NW
    n_chunks = b_per_w // _CHUNK
    chunk_words = _CHUNK * _D
    mesh = plsc.VectorSubcoreMesh(core_axis_name="c", subcore_axis_name="s")

    @functools.partial(
        pl.kernel,
        out_type=jax.ShapeDtypeStruct((B * _D,), jnp.float32),
        mesh=mesh,
        scratch_types=[
            pltpu.VMEM((n_chunks, _CHUNK), jnp.int32),
            pltpu.VMEM((_CHUNK, _DP), jnp.float32),
            pltpu.VMEM((chunk_words + 16,), jnp.float32),
            pltpu.SemaphoreType.DMA,
        ],
        compiler_params=pltpu.CompilerParams(use_tc_tiling_on_sc=False),
    )
    def gather(steps_hbm, table_hbm, out_hbm, idx_v, rows_v, cmp_v, sem):
        wid = lax.axis_index("s") * NC + lax.axis_index("c")
        base = wid * b_per_w
        pltpu.sync_copy(steps_hbm.at[pl.ds(wid * n_chunks, n_chunks)], idx_v)

        def chunk_body(c, carry):
            off = pl.multiple_of(c * _CHUNK, _CHUNK)
            pltpu.async_copy(table_hbm.at[idx_v.at[c]], rows_v, sem).wait()

            def row_body(r, carry2):
                row = rows_v.at[r]
                dst = r * _D
                for k in range(0, 96, 16):
                    cmp_v[pl.ds(dst + k, 16)] = row[pl.ds(k, 16)]
                cmp_v[pl.ds(dst + 96, 16)] = row[pl.ds(96, 16)]
                return carry2

            lax.fori_loop(0, _CHUNK, row_body, 0)
            pltpu.sync_copy(
                cmp_v.at[pl.ds(0, chunk_words)],
                out_hbm.at[pl.ds(base * _D + off * _D, chunk_words)],
            )
            return carry

        lax.fori_loop(0, n_chunks, chunk_body, 0)

    return gather


def kernel(steps, embedding):
    B = steps.shape[0] * steps.shape[1]
    V, D = embedding.shape
    flat = steps.reshape(B // _CHUNK, _CHUNK)
    emb_p = jnp.pad(embedding, ((0, 0), (0, _DP - D)))
    out = _build_gather(B, V)(flat, emb_p)
    return out.reshape(steps.shape[0], steps.shape[1], D)


# speedup vs baseline: 1.2942x; 1.2942x over previous
import functools

import jax
import jax.numpy as jnp
from jax import lax
from jax.experimental import pallas as pl
from jax.experimental.pallas import tpu as pltpu
from jax.experimental.pallas import tpu_sc as plsc

_D = 110
_DP = 112
_CHUNK = 128
_GPS = 2
_R = _CHUNK * _GPS


@functools.lru_cache(maxsize=None)
def _build_gather(B: int, V: int):
    info = plsc.get_sparse_core_info()
    NC, NS = info.num_cores, info.num_subcores
    NW = NC * NS
    assert B % (NW * _R) == 0
    b_per_w = B // NW
    n_chunks = b_per_w // _CHUNK
    n_super = b_per_w // _R
    sc_words = _R * _D
    mesh = plsc.VectorSubcoreMesh(core_axis_name="c", subcore_axis_name="s")

    @functools.partial(
        pl.kernel,
        out_type=jax.ShapeDtypeStruct((B * _D,), jnp.float32),
        mesh=mesh,
        scratch_types=[
            pltpu.VMEM((n_chunks, _CHUNK), jnp.int32),
            pltpu.VMEM((2, _R, _DP), jnp.float32),
            pltpu.VMEM((2, sc_words + 16), jnp.float32),
            pltpu.SemaphoreType.DMA,
            pltpu.SemaphoreType.DMA,
            pltpu.SemaphoreType.DMA,
            pltpu.SemaphoreType.DMA,
        ],
        compiler_params=pltpu.CompilerParams(use_tc_tiling_on_sc=False),
    )
    def gather(steps_hbm, table_hbm, out_hbm, idx_v, pad_v, cmp_v, g0, g1, w0, w1):
        sem_g = (g0, g1)
        sem_w = (w0, w1)
        wid = lax.axis_index("s") * NC + lax.axis_index("c")
        base = wid * b_per_w
        pltpu.sync_copy(steps_hbm.at[pl.ds(wid * n_chunks, n_chunks)], idx_v)

        def gather_desc(t, b, i):
            c = t * _GPS + i
            return pltpu.make_async_copy(
                table_hbm.at[idx_v.at[c]],
                pad_v.at[b].at[pl.ds(i * _CHUNK, _CHUNK)],
                sem_g[b],
            )

        def write_desc(t, b):
            return pltpu.make_async_copy(
                cmp_v.at[b].at[pl.ds(0, sc_words)],
                out_hbm.at[pl.ds(base * _D + t * sc_words, sc_words)],
                sem_w[b],
            )

        def compact(b):
            src = pad_v.at[b]
            dst = cmp_v.at[b]

            def row_body(r, carry2):
                row = src.at[r]
                d = r * _D
                for k in range(0, _DP - 16, 16):
                    dst[pl.ds(d + k, 16)] = row[pl.ds(k, 16)]
                dst[pl.ds(d + _DP - 16, 16)] = row[pl.ds(_DP - 16, 16)]
                return carry2

            lax.fori_loop(0, _R, row_body, 0)

        for b in (0, 1):
            for i in range(_GPS):
                gather_desc(b, b, i).start()

        def pair_body(u, carry):
            for b in (0, 1):
                t = 2 * u + b
                for i in range(_GPS):
                    gather_desc(t, b, i).wait()

                @pl.when(t >= 2)
                def _():
                    write_desc(t - 2, b).wait()

                compact(b)
                write_desc(t, b).start()

                @pl.when(t + 2 < n_super)
                def _():
                    for i in range(_GPS):
                        gather_desc(t + 2, b, i).start()
            return carry

        lax.fori_loop(0, n_super // 2, pair_body, 0)
        for b in (0, 1):
            write_desc(n_super - 2 + b, b).wait()

    return gather


def kernel(steps, embedding):
    B = steps.shape[0] * steps.shape[1]
    V, D = embedding.shape
    flat = steps.reshape(B // _CHUNK, _CHUNK)
    emb_p = jnp.pad(embedding, ((0, 0), (0, _DP - D)))
    out = _build_gather(B, V)(flat, emb_p)
    return out.reshape(steps.shape[0], steps.shape[1], D)
